# Initial kernel scaffold; baseline (speedup 1.0000x reference)
import jax
import jax.numpy as jnp
from jax.experimental import pallas as pl


def kernel(x, lap_indices, lap_values, g1, be1, W1, b1, g2, be2, W2, b2):
    raise NotImplementedError("write your pallas kernel here")


# speedup vs baseline: 5.5790x; 5.5790x over previous
import functools

import jax
import jax.numpy as jnp
from jax import lax
from jax.experimental import pallas as pl
from jax.experimental.pallas import tpu as pltpu
from jax.experimental.pallas import tpu_sc as plsc

N = 10000
E = 320000
D = 128
EPS = 1e-5

NC = 2
NS = 16
NW = NC * NS
EDGES_PER_TILE = E // NW
CHUNK = 80
NCHUNK = EDGES_PER_TILE // CHUNK
HALF = 64
NP = 10240
ROWS_PER_TILE = NP // NS
LANES = 16
DSTEP = D // LANES


def _spmm_body(x_hbm, src_hbm, dst_hbm, val_hbm, out_hbm,
               src_v, dst_v, val_v, rows_v, acc_sh, gsem):
    c = lax.axis_index("c")
    s = lax.axis_index("s")
    wid = c * NS + s

    def _zrow(r, carry):
        for j in range(DSTEP):
            rows_v[r, pl.ds(j * LANES, LANES)] = jnp.zeros((LANES,), jnp.float32)
        return carry
    lax.fori_loop(0, CHUNK, _zrow, 0)
    for t in range(ROWS_PER_TILE // CHUNK):
        pltpu.sync_copy(
            rows_v, acc_sh.at[pl.ds(s * ROWS_PER_TILE + t * CHUNK, CHUNK)])
    plsc.subcore_barrier()

    for koff, npass in ((0, HALF), (HALF, NCHUNK - HALF)):
        pltpu.sync_copy(src_hbm.at[wid, pl.ds(koff, npass)],
                        src_v.at[pl.ds(0, npass)])
        pltpu.sync_copy(dst_hbm.at[wid, pl.ds(koff, npass)],
                        dst_v.at[pl.ds(0, npass)])
        pltpu.sync_copy(val_hbm.at[wid, pl.ds(koff, npass)],
                        val_v.at[pl.ds(0, npass)])

        def _chunk(k, carry):
            pltpu.async_copy(x_hbm.at[src_v.at[k]], rows_v, gsem).wait()
            def _group(g, carry2):
                vv = val_v[k, pl.ds(g * LANES, LANES)]
                for e in range(LANES):
                    v = vv[e]
                    i = g * LANES + e
                    for j in range(DSTEP):
                        sl = pl.ds(j * LANES, LANES)
                        rows_v[i, sl] = rows_v[i, sl] * v
                return carry2
            lax.fori_loop(0, CHUNK // LANES, _group, 0)
            pltpu.sync_copy(rows_v, acc_sh.at[dst_v.at[k]], add=True)
            return carry
        lax.fori_loop(0, npass, _chunk, 0)

    plsc.subcore_barrier()
    for t in range(ROWS_PER_TILE // CHUNK):
        base = s * ROWS_PER_TILE + t * CHUNK
        pltpu.sync_copy(acc_sh.at[pl.ds(base, CHUNK)], rows_v)
        pltpu.sync_copy(rows_v, out_hbm.at[c, pl.ds(base, CHUNK)])


_spmm_call = functools.partial(
    pl.kernel,
    out_type=jax.ShapeDtypeStruct((NC, NP, D), jnp.float32),
    mesh=plsc.VectorSubcoreMesh(core_axis_name="c", subcore_axis_name="s"),
    scratch_types=[
        pltpu.VMEM((HALF, CHUNK), jnp.int32),
        pltpu.VMEM((HALF, CHUNK), jnp.int32),
        pltpu.VMEM((HALF, CHUNK), jnp.float32),
        pltpu.VMEM((CHUNK, D), jnp.float32),
        pltpu.VMEM_SHARED((NP, D), jnp.float32),
        pltpu.SemaphoreType.DMA,
    ],
)(_spmm_body)


def _spmm(x, src2d, dst2d, val2d):
    return _spmm_call(x, src2d, dst2d, val2d)


def _bn_body(x_ref, g_ref, b_ref, o_ref):
    x = x_ref[...]
    mu = jnp.mean(x, axis=0, keepdims=True)
    xc = x - mu
    var = jnp.mean(xc * xc, axis=0, keepdims=True)
    o_ref[...] = g_ref[...] * (xc * lax.rsqrt(var + EPS)) + b_ref[...]


def _combine_body(p_ref, o_ref):
    o_ref[...] = p_ref[0, :N, :] + p_ref[1, :N, :]


def _mid_body(xn_ref, y1_ref, y2p_ref, w_ref, b_ref, g_ref, be_ref, o_ref):
    wa = w_ref[0] - w_ref[2]
    wb = w_ref[1]
    wc = 2.0 * w_ref[2]
    o = (jnp.dot(xn_ref[...], wa, preferred_element_type=jnp.float32)
         + jnp.dot(y1_ref[...], wb, preferred_element_type=jnp.float32)
         + jnp.dot(y2p_ref[0, :N, :] + y2p_ref[1, :N, :], wc, preferred_element_type=jnp.float32)
         + b_ref[...])
    o = jnp.maximum(o, 0.0)
    mu = jnp.mean(o, axis=0, keepdims=True)
    oc = o - mu
    var = jnp.mean(oc * oc, axis=0, keepdims=True)
    o_ref[...] = g_ref[...] * (oc * lax.rsqrt(var + EPS)) + be_ref[...]


def _final_body(z_ref, y3_ref, y4p_ref, w_ref, b_ref, xn_ref, o_ref):
    wa = w_ref[0] - w_ref[2]
    wb = w_ref[1]
    wc = 2.0 * w_ref[2]
    o = (jnp.dot(z_ref[...], wa, preferred_element_type=jnp.float32)
         + jnp.dot(y3_ref[...], wb, preferred_element_type=jnp.float32)
         + jnp.dot(y4p_ref[0, :N, :] + y4p_ref[1, :N, :], wc, preferred_element_type=jnp.float32)
         + b_ref[...])
    o_ref[...] = jnp.maximum(o + xn_ref[...], 0.0)


_ND = jax.ShapeDtypeStruct((N, D), jnp.float32)


def kernel(x, lap_indices, lap_values, g1, be1, W1, b1, g2, be2, W2, b2):
    src2d = lap_indices[1].reshape(NW, NCHUNK, CHUNK)
    dst2d = lap_indices[0].reshape(NW, NCHUNK, CHUNK)
    val2d = lap_values.reshape(NW, NCHUNK, CHUNK)
    g1r, be1r, b1r = g1.reshape(1, D), be1.reshape(1, D), b1.reshape(1, D)
    g2r, be2r, b2r = g2.reshape(1, D), be2.reshape(1, D), b2.reshape(1, D)

    xn = pl.pallas_call(_bn_body, out_shape=_ND)(x, g1r, be1r)
    y1p = _spmm(xn, src2d, dst2d, val2d)
    y1 = pl.pallas_call(_combine_body, out_shape=_ND)(y1p)
    y2p = _spmm(y1, src2d, dst2d, val2d)
    z = pl.pallas_call(_mid_body, out_shape=_ND)(
        xn, y1, y2p, W1, b1r, g2r, be2r)
    y3p = _spmm(z, src2d, dst2d, val2d)
    y3 = pl.pallas_call(_combine_body, out_shape=_ND)(y3p)
    y4p = _spmm(y3, src2d, dst2d, val2d)
    out = pl.pallas_call(_final_body, out_shape=_ND)(
        z, y3, y4p, W2, b2r, xn)
    return out


# speedup vs baseline: 9.7357x; 1.7451x over previous
import functools

import jax
import jax.numpy as jnp
from jax import lax
from jax.experimental import pallas as pl
from jax.experimental.pallas import tpu as pltpu
from jax.experimental.pallas import tpu_sc as plsc

N = 10000
E = 320000
D = 128
EPS = 1e-5

NC = 2
NS = 16
NW = NC * NS
EDGES_PER_TILE = E // NW
CHUNK = 80
NCHUNK = EDGES_PER_TILE // CHUNK
PASS = 32
NP = 10240
ROWS_PER_TILE = NP // NS
LANES = 16
DSTEP = D // LANES


def _spmm_body(x_hbm, src_hbm, dst_hbm, val_hbm, out_hbm,
               src_v, dst_v, val_v, rows0_v, rows1_v, rows2_v,
               acc_sh, gsem0, gsem1, gsem2, ssem0, ssem1, ssem2):
    c = lax.axis_index("c")
    s = lax.axis_index("s")
    wid = c * NS + s
    rows = (rows0_v, rows1_v, rows2_v)
    gsem = (gsem0, gsem1, gsem2)
    ssem = (ssem0, ssem1, ssem2)

    def _start_gather(k, b):
        pltpu.async_copy(x_hbm.at[src_v.at[k]], rows[b], gsem[b])

    def _wait_gather(k, b):
        pltpu.make_async_copy(x_hbm.at[src_v.at[k]], rows[b], gsem[b]).wait()

    def _start_scatter(k, b):
        pltpu.async_copy(rows[b], acc_sh.at[dst_v.at[k]], ssem[b], add=True)

    def _wait_scatter(k, b):
        pltpu.make_async_copy(rows[b], acc_sh.at[dst_v.at[k]], ssem[b]).wait()

    def _scale(k, b):
        rb = rows[b]
        def _group(g, carry2):
            vv = val_v[k, pl.ds(g * LANES, LANES)]
            for e in range(LANES):
                v = vv[e]
                i = g * LANES + e
                for j in range(DSTEP):
                    sl = pl.ds(j * LANES, LANES)
                    rb[i, sl] = rb[i, sl] * v
            return carry2
        lax.fori_loop(0, CHUNK // LANES, _group, 0)

    def _zrow(r, carry):
        for j in range(DSTEP):
            rows0_v[r, pl.ds(j * LANES, LANES)] = jnp.zeros((LANES,), jnp.float32)
        return carry
    lax.fori_loop(0, CHUNK, _zrow, 0)
    for t in range(ROWS_PER_TILE // CHUNK):
        pltpu.sync_copy(
            rows0_v, acc_sh.at[pl.ds(s * ROWS_PER_TILE + t * CHUNK, CHUNK)])
    plsc.subcore_barrier()

    for koff in range(0, NCHUNK, PASS):
        npass = min(PASS, NCHUNK - koff)
        pltpu.sync_copy(src_hbm.at[wid, pl.ds(koff, npass)],
                        src_v.at[pl.ds(0, npass)])
        pltpu.sync_copy(dst_hbm.at[wid, pl.ds(koff, npass)],
                        dst_v.at[pl.ds(0, npass)])
        pltpu.sync_copy(val_hbm.at[wid, pl.ds(koff, npass)],
                        val_v.at[pl.ds(0, npass)])

        _start_gather(0, 0)
        ntrip = npass // 3

        def _triple(t, carry):
            for b in range(3):
                k = 3 * t + b
                nb = (b + 1) % 3
                if b == 0:
                    @pl.when(t >= 1)
                    def _():
                        _wait_scatter(k - 2, nb)
                    _start_gather(k + 1, nb)
                elif b == 1:
                    @pl.when(t >= 1)
                    def _():
                        _wait_scatter(k - 2, nb)
                    _start_gather(k + 1, nb)
                else:
                    @pl.when(3 * t + 3 < npass)
                    def _():
                        _wait_scatter(k - 2, nb)
                        _start_gather(k + 1, nb)
                _wait_gather(k, b)
                _scale(k, b)
                _start_scatter(k, b)
            return carry
        lax.fori_loop(0, ntrip, _triple, 0)

        for k in range(3 * ntrip, npass):
            b = k % 3
            if k + 1 < npass:
                _wait_scatter(k - 2, (b + 1) % 3)
                _start_gather(k + 1, (b + 1) % 3)
            _wait_gather(k, b)
            _scale(k, b)
            _start_scatter(k, b)
        for k in range(max(0, npass - 3), npass):
            _wait_scatter(k, k % 3)

    plsc.subcore_barrier()
    for t in range(ROWS_PER_TILE // CHUNK):
        base = s * ROWS_PER_TILE + t * CHUNK
        pltpu.sync_copy(acc_sh.at[pl.ds(base, CHUNK)], rows0_v)
        pltpu.sync_copy(rows0_v, out_hbm.at[c, pl.ds(base, CHUNK)])


_spmm_call = functools.partial(
    pl.kernel,
    out_type=jax.ShapeDtypeStruct((NC, NP, D), jnp.float32),
    mesh=plsc.VectorSubcoreMesh(core_axis_name="c", subcore_axis_name="s"),
    scratch_types=[
        pltpu.VMEM((PASS, CHUNK), jnp.int32),
        pltpu.VMEM((PASS, CHUNK), jnp.int32),
        pltpu.VMEM((PASS, CHUNK), jnp.float32),
        pltpu.VMEM((CHUNK, D), jnp.float32),
        pltpu.VMEM((CHUNK, D), jnp.float32),
        pltpu.VMEM((CHUNK, D), jnp.float32),
        pltpu.VMEM_SHARED((NP, D), jnp.float32),
        pltpu.SemaphoreType.DMA,
        pltpu.SemaphoreType.DMA,
        pltpu.SemaphoreType.DMA,
        pltpu.SemaphoreType.DMA,
        pltpu.SemaphoreType.DMA,
        pltpu.SemaphoreType.DMA,
    ],
)(_spmm_body)


def _spmm(x, src2d, dst2d, val2d):
    return _spmm_call(x, src2d, dst2d, val2d)


def _bn_body(x_ref, g_ref, b_ref, o_ref):
    x = x_ref[...]
    mu = jnp.mean(x, axis=0, keepdims=True)
    xc = x - mu
    var = jnp.mean(xc * xc, axis=0, keepdims=True)
    o_ref[...] = g_ref[...] * (xc * lax.rsqrt(var + EPS)) + b_ref[...]


def _combine_body(p_ref, o_ref):
    o_ref[...] = p_ref[0, :N, :] + p_ref[1, :N, :]


def _mid_body(xn_ref, y1_ref, y2p_ref, w_ref, b_ref, g_ref, be_ref, o_ref):
    wa = w_ref[0] - w_ref[2]
    wb = w_ref[1]
    wc = 2.0 * w_ref[2]
    o = (jnp.dot(xn_ref[...], wa, preferred_element_type=jnp.float32)
         + jnp.dot(y1_ref[...], wb, preferred_element_type=jnp.float32)
         + jnp.dot(y2p_ref[0, :N, :] + y2p_ref[1, :N, :], wc, preferred_element_type=jnp.float32)
         + b_ref[...])
    o = jnp.maximum(o, 0.0)
    mu = jnp.mean(o, axis=0, keepdims=True)
    oc = o - mu
    var = jnp.mean(oc * oc, axis=0, keepdims=True)
    o_ref[...] = g_ref[...] * (oc * lax.rsqrt(var + EPS)) + be_ref[...]


def _final_body(z_ref, y3_ref, y4p_ref, w_ref, b_ref, xn_ref, o_ref):
    wa = w_ref[0] - w_ref[2]
    wb = w_ref[1]
    wc = 2.0 * w_ref[2]
    o = (jnp.dot(z_ref[...], wa, preferred_element_type=jnp.float32)
         + jnp.dot(y3_ref[...], wb, preferred_element_type=jnp.float32)
         + jnp.dot(y4p_ref[0, :N, :] + y4p_ref[1, :N, :], wc, preferred_element_type=jnp.float32)
         + b_ref[...])
    o_ref[...] = jnp.maximum(o + xn_ref[...], 0.0)


_ND = jax.ShapeDtypeStruct((N, D), jnp.float32)


def kernel(x, lap_indices, lap_values, g1, be1, W1, b1, g2, be2, W2, b2):
    src2d = lap_indices[1].reshape(NW, NCHUNK, CHUNK)
    dst2d = lap_indices[0].reshape(NW, NCHUNK, CHUNK)
    val2d = lap_values.reshape(NW, NCHUNK, CHUNK)
    g1r, be1r, b1r = g1.reshape(1, D), be1.reshape(1, D), b1.reshape(1, D)
    g2r, be2r, b2r = g2.reshape(1, D), be2.reshape(1, D), b2.reshape(1, D)

    xn = pl.pallas_call(_bn_body, out_shape=_ND)(x, g1r, be1r)
    y1p = _spmm(xn, src2d, dst2d, val2d)
    y1 = pl.pallas_call(_combine_body, out_shape=_ND)(y1p)
    y2p = _spmm(y1, src2d, dst2d, val2d)
    z = pl.pallas_call(_mid_body, out_shape=_ND)(
        xn, y1, y2p, W1, b1r, g2r, be2r)
    y3p = _spmm(z, src2d, dst2d, val2d)
    y3 = pl.pallas_call(_combine_body, out_shape=_ND)(y3p)
    y4p = _spmm(y3, src2d, dst2d, val2d)
    out = pl.pallas_call(_final_body, out_shape=_ND)(
        z, y3, y4p, W2, b2r, xn)
    return out
